# baseline (speedup 1.0000x reference)
import functools

import jax
import jax.numpy as jnp
from jax.experimental import pallas as pl
from jax.experimental.pallas import tpu as pltpu

TOP_K = 48
NUM_RBF = 16
MAX_REL = 32
NUM_POS = 16
EDGE_FEATURES = 128

BLK_Q = 128
CHUNK_Q = 32
NUM_IN = NUM_POS + NUM_RBF * 25
NP = 24
TW = 3 * NP + 2

_PAIRS = [("N", "N"), ("C", "C"), ("O", "O"), ("Cb", "Cb"), ("Ca", "N"),
          ("Ca", "C"), ("Ca", "O"), ("Ca", "Cb"), ("N", "C"), ("N", "O"),
          ("N", "Cb"), ("Cb", "C"), ("Cb", "O"), ("O", "C"), ("N", "Ca"),
          ("C", "Ca"), ("O", "Ca"), ("Cb", "Ca"), ("C", "N"), ("O", "N"),
          ("Cb", "N"), ("C", "Cb"), ("O", "Cb"), ("C", "O")]
_CA_A = _PAIRS.index(("Ca", "N"))
_CA_B = _PAIRS.index(("N", "Ca"))


def _fiota(shape, dim):
    return jax.lax.broadcasted_iota(jnp.int32, shape, dim).astype(jnp.float32)


def _features_kernel(ta_ref, tb_ref, qa_ref, mask_ref, mq_ref, wpos_ref,
                     bpos_ref, wedge_ref, lns_ref, lno_ref, e_ref, idx_ref,
                     *, L):
    f32 = jnp.float32
    HI = jax.lax.Precision.HIGHEST
    tb = tb_ref[0]
    qa = qa_ref[0]
    mk = mask_ref[0]
    mq = mq_ref[0]

    lane_l = _fiota((BLK_Q, L), 1)

    d2 = jnp.zeros((BLK_Q, L), f32)
    for c in range(3):
        qc = qa[:, _CA_A + c * NP:_CA_A + c * NP + 1]
        kc = tb[:, _CA_B + c * NP:_CA_B + c * NP + 1].reshape(1, L)
        d2 = d2 + (qc - kc) ** 2
    m2d = mq.reshape(BLK_Q, 1) * mk
    dist = m2d * jnp.sqrt(d2 + 1e-06)
    dmax = jnp.max(dist, axis=1, keepdims=True)
    dadj = dist + (1.0 - m2d) * dmax

    BIG = f32(3e38)
    cur = dadj
    dn_cols = []
    ix_cols = []
    for _ in range(TOP_K):
        m = jnp.min(cur, axis=1, keepdims=True)
        eq = cur == m
        ix = jnp.min(jnp.where(eq, lane_l, f32(L)), axis=1, keepdims=True)
        cur = jnp.where(lane_l == ix, BIG, cur)
        dn_cols.append(m)
        ix_cols.append(ix)
    dn = jnp.concatenate(dn_cols, axis=1)
    ixf = jnp.concatenate(ix_cols, axis=1)
    idx_ref[0] = ixf.astype(jnp.int32)

    mu = 2.0 + _fiota((1, NUM_RBF), 1) * (20.0 / 15.0)
    mu384 = 2.0 + (jax.lax.broadcasted_iota(jnp.int32, (1, NP * NUM_RBF), 1)
                   % NUM_RBF).astype(f32) * (20.0 / 15.0)
    sig = (22.0 - 2.0) / NUM_RBF

    n_chunk = BLK_Q // CHUNK_Q
    R = CHUNK_Q * TOP_K
    sub_r = _fiota((R, 1), 0)
    qid_rel = jnp.floor(sub_r / TOP_K)
    kid = sub_r - qid_rel * TOP_K
    ohk = (_fiota((R, TOP_K), 1) == kid).astype(f32)
    lane_q = _fiota((R, BLK_Q), 1)
    lane_tbl = _fiota((R, L), 1)
    lane_pos = _fiota((R, 2 * MAX_REL + 2), 1)

    for c in range(n_chunk):
        ohq = (lane_q == qid_rel + f32(c * CHUNK_Q)).astype(f32)
        pik = jnp.dot(ohq, jnp.concatenate([ixf, dn], axis=1),
                      preferred_element_type=f32, precision=HI)
        fidx = jnp.sum(pik[:, :TOP_K] * ohk, axis=1, keepdims=True)
        fdn = jnp.sum(pik[:, TOP_K:] * ohk, axis=1, keepdims=True)
        oht = (lane_tbl == fidx).astype(f32)
        gb = jnp.dot(oht, tb, preferred_element_type=f32, precision=HI)
        ga = jnp.dot(ohq, qa, preferred_element_type=f32, precision=HI)

        pd2 = ((ga[:, 0:NP] - gb[:, 0:NP]) ** 2
               + (ga[:, NP:2 * NP] - gb[:, NP:2 * NP]) ** 2
               + (ga[:, 2 * NP:3 * NP] - gb[:, 2 * NP:3 * NP]) ** 2)
        pdist = jnp.sqrt(pd2 + 1e-06)
        d384 = jnp.broadcast_to(pdist[:, :, None], (R, NP, NUM_RBF))
        d384 = d384.reshape(R, NP * NUM_RBF)
        rbf384 = jnp.exp(-(((d384 - mu384) / sig) ** 2))

        off = ga[:, 3 * NP:3 * NP + 1] - gb[:, 3 * NP:3 * NP + 1]
        ech = (ga[:, 3 * NP + 1:] == gb[:, 3 * NP + 1:]).astype(f32)
        dpos = jnp.clip(off + MAX_REL, 0.0, 2.0 * MAX_REL) * ech \
            + (1.0 - ech) * (2.0 * MAX_REL + 1.0)
        ohd = (lane_pos == dpos).astype(f32)
        epos = jnp.dot(ohd, wpos_ref[:], preferred_element_type=f32,
                       precision=HI) + bpos_ref[:]
        rbfdn = jnp.exp(-(((fdn - mu) / sig) ** 2))

        ef = jnp.concatenate([epos, rbfdn, rbf384], axis=1)
        e = jnp.dot(ef, wedge_ref[:], preferred_element_type=f32,
                    precision=HI)
        emu = jnp.mean(e, axis=1, keepdims=True)
        ec = e - emu
        var = jnp.mean(ec * ec, axis=1, keepdims=True)
        e = ec / jnp.sqrt(var + 1e-05) * lns_ref[:] + lno_ref[:]
        e_ref[0, c * R:(c + 1) * R, :] = e


@functools.partial(jax.jit, static_argnames=())
def kernel(X, mask, residue_idx, chain_labels, W_pos, b_pos, W_edge,
           ln_scale, ln_offset):
    B, L = mask.shape
    bvec = X[:, :, 1, :] - X[:, :, 0, :]
    cvec = X[:, :, 2, :] - X[:, :, 1, :]
    avec = jnp.cross(bvec, cvec)
    Cb = -0.58273431 * avec + 0.56802827 * bvec - 0.54067466 * cvec \
        + X[:, :, 1, :]
    atoms = {"N": X[:, :, 0, :], "Ca": X[:, :, 1, :], "C": X[:, :, 2, :],
             "O": X[:, :, 3, :], "Cb": Cb}
    meta = [residue_idx[:, :, None].astype(jnp.float32),
            chain_labels[:, :, None].astype(jnp.float32)]
    ta = jnp.concatenate(
        [jnp.stack([atoms[a][:, :, c] for a, _ in _PAIRS], axis=-1)
         for c in range(3)] + meta, axis=-1)
    tbm = jnp.concatenate(
        [jnp.stack([atoms[b][:, :, c] for _, b in _PAIRS], axis=-1)
         for c in range(3)] + meta, axis=-1)

    nblk = L // BLK_Q
    grid = (B, nblk)
    kfn = functools.partial(_features_kernel, L=L)
    e_flat, e_idx = pl.pallas_call(
        kfn,
        grid=grid,
        compiler_params=pltpu.CompilerParams(
            dimension_semantics=("parallel", "parallel")),
        in_specs=[
            pl.BlockSpec((1, L, TW), lambda b, r: (b, 0, 0)),
            pl.BlockSpec((1, L, TW), lambda b, r: (b, 0, 0)),
            pl.BlockSpec((1, BLK_Q, TW), lambda b, r: (b, r, 0)),
            pl.BlockSpec((1, 1, L), lambda b, r: (b, 0, 0)),
            pl.BlockSpec((1, 1, BLK_Q), lambda b, r: (b, 0, r)),
            pl.BlockSpec((2 * MAX_REL + 2, NUM_POS), lambda b, r: (0, 0)),
            pl.BlockSpec((1, NUM_POS), lambda b, r: (0, 0)),
            pl.BlockSpec((NUM_IN, EDGE_FEATURES), lambda b, r: (0, 0)),
            pl.BlockSpec((1, EDGE_FEATURES), lambda b, r: (0, 0)),
            pl.BlockSpec((1, EDGE_FEATURES), lambda b, r: (0, 0)),
        ],
        out_specs=[
            pl.BlockSpec((1, BLK_Q * TOP_K, EDGE_FEATURES),
                         lambda b, r: (b, r, 0)),
            pl.BlockSpec((1, BLK_Q, TOP_K), lambda b, r: (b, r, 0)),
        ],
        out_shape=[
            jax.ShapeDtypeStruct((B, L * TOP_K, EDGE_FEATURES), jnp.float32),
            jax.ShapeDtypeStruct((B, L, TOP_K), jnp.int32),
        ],
    )(ta, tbm, ta, mask[:, None, :], mask[:, None, :], W_pos, b_pos[None, :],
      W_edge, ln_scale[None, :], ln_offset[None, :])
    return e_flat.reshape(B, L, TOP_K, EDGE_FEATURES), e_idx
